# baseline (speedup 1.0000x reference)
import functools

import jax
import jax.numpy as jnp
from jax import lax
from jax.experimental import pallas as pl
from jax.experimental.pallas import tpu as pltpu
from jax.experimental.pallas import tpu_sc as plsc


def _make_tc_detile(V, D):
    CH = 4096
    G = -(-V // CH)
    RB = CH * D // 128
    RT = V * D // 128

    def tkern(x_ref, o_ref):
        o_ref[...] = x_ref[...].T.reshape(RB, 128)

    return pl.pallas_call(
        tkern,
        grid=(G,),
        in_specs=[pl.BlockSpec((D, CH), lambda i: (0, i))],
        out_specs=pl.BlockSpec((RB, 128), lambda i: (i, 0)),
        out_shape=jax.ShapeDtypeStruct((RT, 128), jnp.float32),
    )


def _make_sc_gather(H, B, V, D):
    info = plsc.get_sparse_core_info()
    NC, NS, L = info.num_cores, info.num_subcores, info.num_lanes
    NW = NC * NS
    CB = B // NW
    BLK = 128
    KG = CB // BLK
    NJ = CB // L
    assert CB * NW == B and KG * BLK == CB and NJ * L == CB
    assert H % 2 == 0

    mesh = plsc.VectorSubcoreMesh(core_axis_name="c", subcore_axis_name="s")

    @functools.partial(
        pl.kernel,
        mesh=mesh,
        out_type=jax.ShapeDtypeStruct((H, D, B), jnp.float32),
        scratch_types=[
            pltpu.VMEM((CB,), jnp.int32),
            pltpu.VMEM((CB,), jnp.int32),
            pltpu.VMEM((CB, D), jnp.float32),
            pltpu.VMEM((CB, D), jnp.float32),
            pltpu.VMEM((D, CB), jnp.float32),
            pltpu.VMEM((D, CB), jnp.float32),
            pltpu.SemaphoreType.DMA,
            pltpu.SemaphoreType.DMA,
            pltpu.SemaphoreType.DMA,
            pltpu.SemaphoreType.DMA,
            pltpu.SemaphoreType.DMA,
            pltpu.SemaphoreType.DMA,
        ],
        compiler_params=pltpu.CompilerParams(use_tc_tiling_on_sc=False),
    )
    def gather_kernel(xp_hbm, table_hbm, out_hbm, idx0, idx1, rows0, rows1,
                      slab0, slab1, isem0, isem1, gsem0, gsem1, osem0, osem1):
        wid = lax.axis_index("s") * NC + lax.axis_index("c")
        b0 = wid * CB

        def fire_idx(h, idx_v, isem):
            pltpu.async_copy(xp_hbm.at[h, pl.ds(b0, CB)], idx_v, isem)

        def wait_idx(idx_v, isem):
            pltpu.make_async_copy(xp_hbm.at[0, pl.ds(b0, CB)], idx_v,
                                  isem).wait()

        def fire_gathers(idx_v, rows_v, gsem):
            for k in range(KG):
                pltpu.async_copy(
                    table_hbm.at[idx_v.at[pl.ds(k * BLK, BLK)]],
                    rows_v.at[pl.ds(k * BLK, BLK)],
                    gsem,
                )

        def drain_gathers(idx_v, rows_v, gsem):
            for k in range(KG):
                pltpu.make_async_copy(
                    table_hbm.at[idx_v.at[pl.ds(k * BLK, BLK)]],
                    rows_v.at[pl.ds(k * BLK, BLK)],
                    gsem,
                ).wait()

        lane = lax.iota(jnp.int32, L)
        perm_idx = {s2: (lane ^ s2)[:, None] for s2 in (1, 2, 4, 8)}
        lane_mask = {s2: (lane & s2) == 0 for s2 in (1, 2, 4, 8)}
        _dn = lax.GatherDimensionNumbers(
            offset_dims=(), collapsed_slice_dims=(0,), start_index_map=(0,))

        def _perm(vec, idx2):
            return lax.gather(vec, idx2, _dn, slice_sizes=(1,),
                              mode=lax.GatherScatterMode.PROMISE_IN_BOUNDS)

        def transpose16(regs):
            r = list(regs)
            for s2 in (1, 2, 4, 8):
                for i in range(L):
                    if i & s2:
                        continue
                    j2 = i | s2
                    a, b = r[i], r[j2]
                    pa = _perm(a, perm_idx[s2])
                    pb = _perm(b, perm_idx[s2])
                    r[i] = jnp.where(lane_mask[s2], a, pb)
                    r[j2] = jnp.where(lane_mask[s2], pa, b)
            return r

        def transpose(rows_v, slab_v):
            def jbody(j, carry):
                row = j * L
                for dh in range(D // L):
                    regs = [rows_v[row + i, pl.ds(dh * L, L)]
                            for i in range(L)]
                    out = transpose16(regs)
                    for i in range(L):
                        slab_v[dh * L + i, pl.ds(row, L)] = out[i]
                return carry
            lax.fori_loop(0, NJ, jbody, 0)

        def fire_slab(h, slab_v, osem):
            pltpu.async_copy(slab_v, out_hbm.at[h, :, pl.ds(b0, CB)], osem)

        def wait_slab(slab_v, osem):
            pltpu.make_async_copy(slab_v, out_hbm.at[0, :, pl.ds(b0, CB)],
                                  osem).wait()

        fire_idx(0, idx0, isem0)
        wait_idx(idx0, isem0)
        fire_gathers(idx0, rows0, gsem0)
        fire_idx(1, idx1, isem1)

        def body(t, carry):
            h0 = t * 2
            h1 = h0 + 1
            drain_gathers(idx0, rows0, gsem0)
            wait_idx(idx1, isem1)
            fire_gathers(idx1, rows1, gsem1)

            @pl.when(t >= 1)
            def _():
                wait_slab(slab0, osem0)
            transpose(rows0, slab0)
            fire_slab(h0, slab0, osem0)

            @pl.when(h0 + 2 < H)
            def _():
                fire_idx(h0 + 2, idx0, isem0)
            drain_gathers(idx1, rows1, gsem1)

            @pl.when(h0 + 2 < H)
            def _():
                wait_idx(idx0, isem0)
                fire_gathers(idx0, rows0, gsem0)
                fire_idx(h0 + 3, idx1, isem1)

            @pl.when(t >= 1)
            def _():
                wait_slab(slab1, osem1)
            transpose(rows1, slab1)
            fire_slab(h1, slab1, osem1)
            return carry

        lax.fori_loop(0, H // 2, body, 0)
        wait_slab(slab0, osem0)
        wait_slab(slab1, osem1)

    return gather_kernel


def kernel(x, table):
    B, H = x.shape
    V, D = table.shape
    fn = _make_sc_gather(H, B, V, D)
    xp = x.astype(jnp.int32).T
    tbl = lax.optimization_barrier(table.reshape(V * D // 128, 128))
    tbl = tbl.reshape(V, D)
    outp = fn(xp, tbl)
    return outp.transpose(2, 0, 1)


# speedup vs baseline: 1.3357x; 1.3357x over previous
import functools

import jax
import jax.numpy as jnp
from jax import lax
from jax.experimental import pallas as pl
from jax.experimental.pallas import tpu as pltpu
from jax.experimental.pallas import tpu_sc as plsc


def _make_tc_detile(V, D):
    CH = 4096
    G = -(-V // CH)
    RB = CH // 4
    Q = 128 // D

    def tkern(x_ref, o_ref):
        o_ref[...] = jnp.concatenate(
            [x_ref[:, k * RB:(k + 1) * RB].T for k in range(Q)], axis=1)

    return pl.pallas_call(
        tkern,
        grid=(G,),
        in_specs=[pl.BlockSpec((D, CH), lambda i: (0, i))],
        out_specs=pl.BlockSpec((RB, 128), lambda i: (i, 0)),
        out_shape=jax.ShapeDtypeStruct((G * RB, 128), jnp.float32),
    )


def _make_sc_gather(H, B, V, D):
    info = plsc.get_sparse_core_info()
    NC, NS, L = info.num_cores, info.num_subcores, info.num_lanes
    NW = NC * NS
    CB = B // NW
    BLK = 128
    KG = CB // BLK
    NJ = CB // L
    assert CB * NW == B and KG * BLK == CB and NJ * L == CB
    assert H % 2 == 0

    mesh = plsc.VectorSubcoreMesh(core_axis_name="c", subcore_axis_name="s")

    @functools.partial(
        pl.kernel,
        mesh=mesh,
        out_type=jax.ShapeDtypeStruct((H, D, B), jnp.float32),
        scratch_types=[
            pltpu.VMEM((CB,), jnp.int32),
            pltpu.VMEM((CB,), jnp.int32),
            pltpu.VMEM((CB, D), jnp.float32),
            pltpu.VMEM((CB, D), jnp.float32),
            pltpu.VMEM((D, CB), jnp.float32),
            pltpu.VMEM((D, CB), jnp.float32),
            pltpu.SemaphoreType.DMA,
            pltpu.SemaphoreType.DMA,
            pltpu.SemaphoreType.DMA,
            pltpu.SemaphoreType.DMA,
            pltpu.SemaphoreType.DMA,
            pltpu.SemaphoreType.DMA,
        ],
        compiler_params=pltpu.CompilerParams(use_tc_tiling_on_sc=False),
    )
    def gather_kernel(xp_hbm, table_hbm, out_hbm, idx0, idx1, rows0, rows1,
                      slab0, slab1, isem0, isem1, gsem0, gsem1, osem0, osem1):
        wid = lax.axis_index("s") * NC + lax.axis_index("c")
        b0 = wid * CB

        def fire_idx(h, idx_v, isem):
            pltpu.async_copy(xp_hbm.at[h, pl.ds(b0, CB)], idx_v, isem)

        def wait_idx(idx_v, isem):
            pltpu.make_async_copy(xp_hbm.at[0, pl.ds(b0, CB)], idx_v,
                                  isem).wait()

        def remap_idx(idx_v):
            def rbody(g, c):
                v = idx_v[pl.ds(g * L, L)]
                idx_v[pl.ds(g * L, L)] = (
                    (v & jnp.int32(-4096))
                    | ((v & jnp.int32(1023)) << 2)
                    | ((v >> 10) & jnp.int32(3)))
                return c
            lax.fori_loop(0, CB // L, rbody, 0)

        def fire_gathers(idx_v, rows_v, gsem):
            for k in range(KG):
                pltpu.async_copy(
                    table_hbm.at[idx_v.at[pl.ds(k * BLK, BLK)]],
                    rows_v.at[pl.ds(k * BLK, BLK)],
                    gsem,
                )

        def drain_gathers(idx_v, rows_v, gsem):
            for k in range(KG):
                pltpu.make_async_copy(
                    table_hbm.at[idx_v.at[pl.ds(k * BLK, BLK)]],
                    rows_v.at[pl.ds(k * BLK, BLK)],
                    gsem,
                ).wait()

        lane = lax.iota(jnp.int32, L)
        perm_idx = {s2: (lane ^ s2)[:, None] for s2 in (1, 2, 4, 8)}
        lane_mask = {s2: (lane & s2) == 0 for s2 in (1, 2, 4, 8)}
        _dn = lax.GatherDimensionNumbers(
            offset_dims=(), collapsed_slice_dims=(0,), start_index_map=(0,))

        def _perm(vec, idx2):
            return lax.gather(vec, idx2, _dn, slice_sizes=(1,),
                              mode=lax.GatherScatterMode.PROMISE_IN_BOUNDS)

        def transpose16(regs):
            r = list(regs)
            for s2 in (1, 2, 4, 8):
                for i in range(L):
                    if i & s2:
                        continue
                    j2 = i | s2
                    a, b = r[i], r[j2]
                    pa = _perm(a, perm_idx[s2])
                    pb = _perm(b, perm_idx[s2])
                    r[i] = jnp.where(lane_mask[s2], a, pb)
                    r[j2] = jnp.where(lane_mask[s2], pa, b)
            return r

        def transpose(rows_v, slab_v):
            def jbody(j, carry):
                row = j * L
                for dh in range(D // L):
                    regs = [rows_v[row + i, pl.ds(dh * L, L)]
                            for i in range(L)]
                    out = transpose16(regs)
                    for i in range(L):
                        slab_v[dh * L + i, pl.ds(row, L)] = out[i]
                return carry
            lax.fori_loop(0, NJ, jbody, 0)

        def fire_slab(h, slab_v, osem):
            pltpu.async_copy(slab_v, out_hbm.at[h, :, pl.ds(b0, CB)], osem)

        def wait_slab(slab_v, osem):
            pltpu.make_async_copy(slab_v, out_hbm.at[0, :, pl.ds(b0, CB)],
                                  osem).wait()

        fire_idx(0, idx0, isem0)
        wait_idx(idx0, isem0)
        remap_idx(idx0)
        fire_gathers(idx0, rows0, gsem0)
        fire_idx(1, idx1, isem1)

        def body(t, carry):
            h0 = t * 2
            h1 = h0 + 1
            drain_gathers(idx0, rows0, gsem0)
            wait_idx(idx1, isem1)
            remap_idx(idx1)
            fire_gathers(idx1, rows1, gsem1)

            @pl.when(t >= 1)
            def _():
                wait_slab(slab0, osem0)
            transpose(rows0, slab0)
            fire_slab(h0, slab0, osem0)

            @pl.when(h0 + 2 < H)
            def _():
                fire_idx(h0 + 2, idx0, isem0)
            drain_gathers(idx1, rows1, gsem1)

            @pl.when(h0 + 2 < H)
            def _():
                wait_idx(idx0, isem0)
                remap_idx(idx0)
                fire_gathers(idx0, rows0, gsem0)
                fire_idx(h0 + 3, idx1, isem1)

            @pl.when(t >= 1)
            def _():
                wait_slab(slab1, osem1)
            transpose(rows1, slab1)
            fire_slab(h1, slab1, osem1)
            return carry

        lax.fori_loop(0, H // 2, body, 0)
        wait_slab(slab0, osem0)
        wait_slab(slab1, osem1)

    return gather_kernel


def kernel(x, table):
    B, H = x.shape
    V, D = table.shape
    fn = _make_sc_gather(H, B, V, D)
    xp = x.astype(jnp.int32).T
    tbl = _make_tc_detile(V, D)(table.T)
    tbl = tbl.reshape(tbl.shape[0] * 128 // D, D)
    outp = fn(xp, tbl)
    return outp.transpose(2, 0, 1)
